# baseline (speedup 1.0000x reference)
import functools

import jax
import jax.numpy as jnp
from jax import lax
from jax.experimental import pallas as pl
from jax.experimental.pallas import tpu as pltpu
from jax.experimental.pallas import tpu_sc as plsc

SINK = 4
OMEGA = 32
K_WINDOWS = 3
START_IDX = 2
K_SEL = K_WINDOWS + START_IDX

_NUM_CORES = 2
_NUM_SUBCORES = 16
_NUM_WORKERS = _NUM_CORES * _NUM_SUBCORES


def _score_select_kernel(att0_ref, att1_ref, att2_ref, att3_ref,
                         win_ref, idx_ref, obs_scr, *, H, S):
    i = pl.program_id(0)
    hb = att0_ref.shape[1]
    part = ((jnp.sum(att0_ref[0], axis=1) + jnp.sum(att1_ref[0], axis=1))
            + (jnp.sum(att2_ref[0], axis=1) + jnp.sum(att3_ref[0], axis=1)))
    for j in range(hb):
        obs_scr[pl.ds(i * hb + j, 1), :] = part[j:j + 1, :]

    @pl.when(i == pl.num_programs(0) - 1)
    def _select():
        _do_select(obs_scr, win_ref, idx_ref, H=H, S=S)


def _do_select(obs_scr, win_ref, idx_ref, *, H, S):
    num_windows = (S - SINK) // OMEGA
    score_end = SINK + num_windows * OMEGA
    kept = SINK + K_SEL * OMEGA + (S - score_end)
    nwin_pad = ((num_windows + 7) // 8) * 8

    obs = obs_scr[:, :]

    t = lax.broadcasted_iota(jnp.int32, (S, nwin_pad), 0)
    w = lax.broadcasted_iota(jnp.int32, (S, nwin_pad), 1)
    member = (t >= SINK) & (t < score_end) & (((t - SINK) // OMEGA) == w)
    win = jnp.dot(obs, member.astype(jnp.float32),
                  preferred_element_type=jnp.float32,
                  precision=lax.Precision.HIGHEST)
    win_ref[:, :] = win[:, :num_windows]

    widx = lax.broadcasted_iota(jnp.int32, (H, nwin_pad), 1)
    work = jnp.where(widx < num_windows, win, -jnp.inf)
    sel = []
    for _ in range(K_SEL):
        m = jnp.max(work, axis=1, keepdims=True)
        idx_j = jnp.min(jnp.where(work == m, widx, jnp.int32(2**30)),
                        axis=1, keepdims=True)
        sel.append(idx_j)
        work = jnp.where(widx == idx_j, -jnp.inf, work)

    for n in range(K_SEL - 1, 0, -1):
        for i in range(n):
            lo = jnp.minimum(sel[i], sel[i + 1])
            hi = jnp.maximum(sel[i], sel[i + 1])
            sel[i], sel[i + 1] = lo, hi

    p = lax.broadcasted_iota(jnp.int32, (H, kept), 1)
    jv = (p - SINK) // OMEGA
    off = (p - SINK) % OMEGA
    selw = jnp.zeros((H, kept), jnp.int32)
    for j in range(K_SEL):
        selw = jnp.where(jv == j, sel[j], selw)
    mid = selw * OMEGA + SINK + off
    keep = jnp.where(p < SINK, p,
                     jnp.where(p < SINK + K_SEL * OMEGA, mid, p + (S - kept)))
    hrow = lax.broadcasted_iota(jnp.int32, (H, kept), 0)
    idx_ref[:, :] = keep + hrow * S


def _gather_body(k_hbm, v_hbm, idx_hbm, outk_hbm, outv_hbm,
                 idx_v, krows, vrows, semk, semv, *, rows_per_w):
    wid = lax.axis_index("s") * _NUM_CORES + lax.axis_index("c")
    base = wid * rows_per_w
    pltpu.sync_copy(idx_hbm.at[pl.ds(base, rows_per_w)], idx_v)
    ck = pltpu.async_copy(k_hbm.at[idx_v], krows, semk)
    cv = pltpu.async_copy(v_hbm.at[idx_v], vrows, semv)
    ck.wait()
    cv.wait()
    pltpu.sync_copy(krows, outk_hbm.at[pl.ds(base, rows_per_w)])
    pltpu.sync_copy(vrows, outv_hbm.at[pl.ds(base, rows_per_w)])


def _make_gather(total_rows, rows_per_w, D):
    mesh = plsc.VectorSubcoreMesh(core_axis_name="c", subcore_axis_name="s")
    return functools.partial(
        pl.kernel,
        mesh=mesh,
        out_type=[jax.ShapeDtypeStruct((total_rows, D), jnp.float32),
                  jax.ShapeDtypeStruct((total_rows, D), jnp.float32)],
        scratch_types=[pltpu.VMEM((rows_per_w,), jnp.int32),
                       pltpu.VMEM((rows_per_w, D), jnp.float32),
                       pltpu.VMEM((rows_per_w, D), jnp.float32),
                       pltpu.SemaphoreType.DMA,
                       pltpu.SemaphoreType.DMA],
    )(functools.partial(_gather_body, rows_per_w=rows_per_w))


def kernel(past_key, past_value, attn_score_cache):
    B, H, S, D = past_key.shape
    Q = attn_score_cache.shape[2]
    num_windows = (S - SINK) // OMEGA
    score_end = SINK + num_windows * OMEGA
    kept = SINK + K_SEL * OMEGA + (S - score_end)

    HB = 2
    win_scores, idx = pl.pallas_call(
        functools.partial(_score_select_kernel, H=H, S=S),
        grid=(H // HB,),
        in_specs=[pl.BlockSpec((1, HB, Q // 4, S), lambda i: (0, i, 0, 0)),
                  pl.BlockSpec((1, HB, Q // 4, S), lambda i: (0, i, 1, 0)),
                  pl.BlockSpec((1, HB, Q // 4, S), lambda i: (0, i, 2, 0)),
                  pl.BlockSpec((1, HB, Q // 4, S), lambda i: (0, i, 3, 0))],
        out_specs=[pl.BlockSpec((H, num_windows), lambda i: (0, 0)),
                   pl.BlockSpec((H, kept), lambda i: (0, 0))],
        out_shape=[jax.ShapeDtypeStruct((H, num_windows), jnp.float32),
                   jax.ShapeDtypeStruct((H, kept), jnp.int32)],
        scratch_shapes=[pltpu.VMEM((H, S), jnp.float32)],
    )(attn_score_cache, attn_score_cache, attn_score_cache, attn_score_cache)

    total_rows = H * kept
    rows_per_w = total_rows // _NUM_WORKERS
    k_tab = past_key.reshape(B * H * S, D)
    v_tab = past_value.reshape(B * H * S, D)
    idx_flat = idx.reshape(total_rows)

    gk, gv = _make_gather(total_rows, rows_per_w, D)(k_tab, v_tab, idx_flat)
    new_k = gk.reshape(B, H, kept, D)
    new_v = gv.reshape(B, H, kept, D)
    return (new_k, new_v, win_scores)


# speedup vs baseline: 1.0202x; 1.0202x over previous
import functools

import jax
import jax.numpy as jnp
from jax import lax
from jax.experimental import pallas as pl
from jax.experimental.pallas import tpu as pltpu
from jax.experimental.pallas import tpu_sc as plsc

SINK = 4
OMEGA = 32
K_WINDOWS = 3
START_IDX = 2
K_SEL = K_WINDOWS + START_IDX

_NUM_CORES = 2
_NUM_SUBCORES = 16
_NUM_WORKERS = _NUM_CORES * _NUM_SUBCORES


def _score_select_kernel(att0_ref, att1_ref, att2_ref, att3_ref,
                         win_ref, idx_ref, obs_scr, *, H, S):
    i = pl.program_id(0)
    hb = att0_ref.shape[1]
    part = ((jnp.sum(att0_ref[0], axis=1) + jnp.sum(att1_ref[0], axis=1))
            + (jnp.sum(att2_ref[0], axis=1) + jnp.sum(att3_ref[0], axis=1)))
    for j in range(hb):
        obs_scr[pl.ds(i * hb + j, 1), :] = part[j:j + 1, :]

    @pl.when(i == pl.num_programs(0) - 1)
    def _select():
        _do_select(obs_scr, win_ref, idx_ref, H=H, S=S)


def _do_select(obs_scr, win_ref, idx_ref, *, H, S):
    num_windows = (S - SINK) // OMEGA
    score_end = SINK + num_windows * OMEGA
    kept = SINK + K_SEL * OMEGA + (S - score_end)
    nwin_pad = ((num_windows + 7) // 8) * 8

    obs = obs_scr[:, :]

    t = lax.broadcasted_iota(jnp.int32, (S, nwin_pad), 0)
    w = lax.broadcasted_iota(jnp.int32, (S, nwin_pad), 1)
    member = (t >= SINK) & (t < score_end) & (((t - SINK) // OMEGA) == w)
    win = jnp.dot(obs, member.astype(jnp.float32),
                  preferred_element_type=jnp.float32,
                  precision=lax.Precision.HIGHEST)
    win_ref[:, :] = win[:, :num_windows]

    widx = lax.broadcasted_iota(jnp.int32, (H, nwin_pad), 1)
    work = jnp.where(widx < num_windows, win, -jnp.inf)
    sel = []
    for _ in range(K_SEL):
        m = jnp.max(work, axis=1, keepdims=True)
        idx_j = jnp.min(jnp.where(work == m, widx, jnp.int32(2**30)),
                        axis=1, keepdims=True)
        sel.append(idx_j)
        work = jnp.where(widx == idx_j, -jnp.inf, work)

    for n in range(K_SEL - 1, 0, -1):
        for i in range(n):
            lo = jnp.minimum(sel[i], sel[i + 1])
            hi = jnp.maximum(sel[i], sel[i + 1])
            sel[i], sel[i + 1] = lo, hi

    p = lax.broadcasted_iota(jnp.int32, (H, kept), 1)
    jv = (p - SINK) // OMEGA
    off = (p - SINK) % OMEGA
    selw = jnp.zeros((H, kept), jnp.int32)
    for j in range(K_SEL):
        selw = jnp.where(jv == j, sel[j], selw)
    mid = selw * OMEGA + SINK + off
    keep = jnp.where(p < SINK, p,
                     jnp.where(p < SINK + K_SEL * OMEGA, mid, p + (S - kept)))
    hrow = lax.broadcasted_iota(jnp.int32, (H, kept), 0)
    idx_ref[:, :] = keep + hrow * S


def _gather_body(k_hbm, v_hbm, idx_hbm, outk_hbm, outv_hbm,
                 idx_v, krows, vrows, semk, semv, *, rows_per_w):
    wid = lax.axis_index("s") * _NUM_CORES + lax.axis_index("c")
    base = wid * rows_per_w
    pltpu.sync_copy(idx_hbm.at[pl.ds(base, rows_per_w)], idx_v)
    ck = pltpu.async_copy(k_hbm.at[idx_v], krows, semk)
    cv = pltpu.async_copy(v_hbm.at[idx_v], vrows, semv)
    ck.wait()
    cv.wait()
    pltpu.sync_copy(krows, outk_hbm.at[pl.ds(base, rows_per_w)])
    pltpu.sync_copy(vrows, outv_hbm.at[pl.ds(base, rows_per_w)])


def _make_gather(total_rows, rows_per_w, D):
    mesh = plsc.VectorSubcoreMesh(core_axis_name="c", subcore_axis_name="s")
    return functools.partial(
        pl.kernel,
        mesh=mesh,
        out_type=[jax.ShapeDtypeStruct((total_rows, D), jnp.float32),
                  jax.ShapeDtypeStruct((total_rows, D), jnp.float32)],
        scratch_types=[pltpu.VMEM((rows_per_w,), jnp.int32),
                       pltpu.VMEM((rows_per_w, D), jnp.float32),
                       pltpu.VMEM((rows_per_w, D), jnp.float32),
                       pltpu.SemaphoreType.DMA,
                       pltpu.SemaphoreType.DMA],
    )(functools.partial(_gather_body, rows_per_w=rows_per_w))


def kernel(past_key, past_value, attn_score_cache):
    B, H, S, D = past_key.shape
    Q = attn_score_cache.shape[2]
    num_windows = (S - SINK) // OMEGA
    score_end = SINK + num_windows * OMEGA
    kept = SINK + K_SEL * OMEGA + (S - score_end)

    HB = 4
    win_scores, idx = pl.pallas_call(
        functools.partial(_score_select_kernel, H=H, S=S),
        grid=(H // HB,),
        in_specs=[pl.BlockSpec((1, HB, Q // 4, S), lambda i: (0, i, 0, 0)),
                  pl.BlockSpec((1, HB, Q // 4, S), lambda i: (0, i, 1, 0)),
                  pl.BlockSpec((1, HB, Q // 4, S), lambda i: (0, i, 2, 0)),
                  pl.BlockSpec((1, HB, Q // 4, S), lambda i: (0, i, 3, 0))],
        out_specs=[pl.BlockSpec((H, num_windows), lambda i: (0, 0)),
                   pl.BlockSpec((H, kept), lambda i: (0, 0))],
        out_shape=[jax.ShapeDtypeStruct((H, num_windows), jnp.float32),
                   jax.ShapeDtypeStruct((H, kept), jnp.int32)],
        scratch_shapes=[pltpu.VMEM((H, S), jnp.float32)],
    )(attn_score_cache, attn_score_cache, attn_score_cache, attn_score_cache)

    total_rows = H * kept
    rows_per_w = total_rows // _NUM_WORKERS
    k_tab = past_key.reshape(B * H * S, D)
    v_tab = past_value.reshape(B * H * S, D)
    idx_flat = idx.reshape(total_rows)

    gk, gv = _make_gather(total_rows, rows_per_w, D)(k_tab, v_tab, idx_flat)
    new_k = gk.reshape(B, H, kept, D)
    new_v = gv.reshape(B, H, kept, D)
    return (new_k, new_v, win_scores)
